# baseline (speedup 1.0000x reference)
import jax
import jax.numpy as jnp
from jax.experimental import pallas as pl
from jax.experimental.pallas import tpu as pltpu

_P = 16
_DIL = (1, 2, 4, 8)
_S = 2048
_C = 8
_T = _S - _P + 1
_N = 4 * _T
_NL = 8192
_PR = 12288
_SXX = 2728.0
_R2T = 0.5


def _win16(o):
    s = o
    c = jnp.zeros_like(o)
    q = o * o
    n = 1
    for _ in range(4):
        s_sh = s[:, n:]
        c_sh = c[:, n:]
        q_sh = q[:, n:]
        m = s.shape[1] - n
        s = s[:, :m] + s_sh
        c = c[:, :m] + c_sh + jnp.float32(n) * s_sh
        q = q[:, :m] + q_sh
        n *= 2
    return s[:, :_T], c[:, :_T], q[:, :_T]


def _pair_cm(sa, ca, qa, sb, cb, qb):
    sumy = sa + sb
    sxy = (ca - 15.5 * sa) + (cb + 0.5 * sb)
    syy = (qa + qb) - sumy * sumy * (1.0 / 32.0)
    r2 = (sxy * sxy) / (_SXX * syy)
    return (r2 >= _R2T).astype(jnp.int32)


def _lane_shift_right(x, sh, fill):
    pad = jnp.full((x.shape[0], sh), fill, x.dtype)
    return jnp.concatenate([pad, x[:, :-sh]], axis=1)


def _kernel(x_ref, w_ref, b_ref, out_ref, wbuf, svbuf, wtmp, svtmp,
            wsem, ssem):
    x = x_ref[0]
    xb = x.astype(jnp.bfloat16).astype(jnp.float32)
    xpad = jnp.pad(xb, ((0, 0), (120, 0)))

    stats = []
    tails = []
    for di, d in enumerate(_DIL):
        o = jnp.zeros((_C, _S), jnp.float32)
        for k in range(_P):
            off = 120 - (15 - k) * d
            wk = w_ref[di, k].astype(jnp.bfloat16).astype(jnp.float32)
            o = o + wk * xpad[:, off:off + _S]
        o = o + b_ref[di]
        oc = o - jnp.mean(o, axis=1, keepdims=True)
        stats.append(_win16(oc))
        tails.append((o[:, _T - 1:], o[:, :_P]))

        pieces = [jnp.pad(o[:, k:], ((0, 0), (0, k))) for k in range(_P)]
        wt = jnp.concatenate(pieces, axis=0)
        wbuf[pl.ds(di * _T, _T), :] = jnp.transpose(wt)[: _T]

    cm_parts = []
    for di in range(4):
        sa, ca, qa = stats[di]
        interior = _pair_cm(sa[:, : _T - 1], ca[:, : _T - 1], qa[:, : _T - 1],
                            sa[:, 1:], ca[:, 1:], qa[:, 1:])
        if di < 3:
            y = jnp.concatenate([tails[di][0], tails[di + 1][1]], axis=1)
            ym = y - jnp.mean(y, axis=1, keepdims=True)
            xm = (jax.lax.broadcasted_iota(jnp.int32, (_C, 2 * _P), 1)
                  .astype(jnp.float32) - 15.5)
            sxy = jnp.sum(ym * xm, axis=1, keepdims=True)
            syy = jnp.sum(ym * ym, axis=1, keepdims=True)
            r2 = (sxy * sxy) / (_SXX * syy)
            bnd = (r2 >= _R2T).astype(jnp.int32)
        else:
            bnd = jnp.zeros((_C, 1), jnp.int32)
        cm_parts.extend([interior, bnd])
    cm = jnp.concatenate(cm_parts, axis=1)
    cm = jnp.pad(cm, ((0, 0), (0, _NL - _N)))

    lane = jax.lax.broadcasted_iota(jnp.int32, (_C, _NL), 1)
    z = jnp.where(cm == 1, -1, lane)
    sh = 1
    for _ in range(13):
        z = jnp.maximum(z, _lane_shift_right(z, sh, -1))
        sh *= 2
    merge = cm * (((lane - z) & 1))
    valid = (1 - _lane_shift_right(merge, 1, 0)) * (lane < _N).astype(jnp.int32)
    s = 1 - valid
    sh = 1
    for _ in range(13):
        s = s + _lane_shift_right(s, sh, 0)
        sh *= 2
    km = jnp.max(jnp.where(valid == 1, s, 0))

    sv = (s << 2) | (merge << 1) | valid
    svbuf[pl.ds(0, _NL), :] = jnp.transpose(sv)

    @pl.when(km > _NL - _N)
    def _zero_far_pad():
        svbuf[pl.ds(_NL, _PR - _NL), :] = jnp.zeros(
            (_PR - _NL, _C), jnp.int32)

    _CH = 1024
    _chunks = [(c * _CH, min(_CH, _N - c * _CH))
               for c in range((_N + _CH - 1) // _CH)]

    def emit(svk8, we, sz, k, r0, add):
        svk = jnp.concatenate([svk8] * 16, axis=1)
        w1 = we[0:sz]
        w2 = we[1:sz + 1]
        mask = jnp.logical_and((svk & 1) == 1, (svk >> 2) == k)
        val = jnp.where((svk & 2) == 2, 0.5 * (w1 + w2), w1)
        t = jnp.where(mask, val, 0.0)
        if add:
            out_ref[0, pl.ds(r0, sz), :] += t
        else:
            out_ref[0, pl.ds(r0, sz), :] = t

    for r0, sz in _chunks:
        we = wbuf[pl.ds(r0, sz + 8), :]
        svk8 = svbuf[pl.ds(r0, sz), :]
        emit(svk8, we, sz, 0, r0, False)

    def body(k, _):
        for r0, sz in _chunks:
            cw = pltpu.make_async_copy(
                wbuf.at[pl.ds(k + r0, sz + 8)],
                wtmp.at[pl.ds(0, sz + 8)], wsem)
            cs = pltpu.make_async_copy(
                svbuf.at[pl.ds(k + r0, sz)],
                svtmp.at[pl.ds(0, sz)], ssem)
            cw.start()
            cs.start()
            cw.wait()
            cs.wait()
            we = wtmp[pl.ds(0, sz + 8), :]
            svk8 = svtmp[pl.ds(0, sz), :]
            emit(svk8, we, sz, k, r0, True)
        return 0

    jax.lax.fori_loop(1, km + 1, body, 0)


@jax.jit
def kernel(seasonal, conv_w, conv_b):
    xt = seasonal.transpose(0, 2, 1)
    out = pl.pallas_call(
        _kernel,
        grid=(32,),
        in_specs=[
            pl.BlockSpec((1, _C, _S), lambda b: (b, 0, 0)),
            pl.BlockSpec(memory_space=pltpu.SMEM),
            pl.BlockSpec(memory_space=pltpu.SMEM),
        ],
        out_specs=pl.BlockSpec((1, _N, 128), lambda b: (b, 0, 0)),
        out_shape=jax.ShapeDtypeStruct((32, _N, 128), jnp.float32),
        scratch_shapes=[
            pltpu.VMEM((_PR, 128), jnp.float32),
            pltpu.VMEM((_PR, _C), jnp.int32),
            pltpu.VMEM((1032, 128), jnp.float32),
            pltpu.VMEM((1024, _C), jnp.int32),
            pltpu.SemaphoreType.DMA,
            pltpu.SemaphoreType.DMA,
        ],
        compiler_params=pltpu.CompilerParams(
            dimension_semantics=("parallel",),
        ),
    )(xt, conv_w, conv_b)
    return out.reshape(32, _N, _P, _C)


# speedup vs baseline: 2.6477x; 2.6477x over previous
import jax
import jax.numpy as jnp
from jax.experimental import pallas as pl
from jax.experimental.pallas import tpu as pltpu

_P = 16
_DIL = (1, 2, 4, 8)
_S = 2048
_C = 8
_T = _S - _P + 1
_N = 4 * _T
_NL = 8192
_PR = 12288
_SXX = 2728.0
_R2T = 0.5


def _win16(o):
    s = o
    c = jnp.zeros_like(o)
    q = o * o
    n = 1
    for _ in range(4):
        s_sh = s[:, n:]
        c_sh = c[:, n:]
        q_sh = q[:, n:]
        m = s.shape[1] - n
        s = s[:, :m] + s_sh
        c = c[:, :m] + c_sh + jnp.float32(n) * s_sh
        q = q[:, :m] + q_sh
        n *= 2
    return s[:, :_T], c[:, :_T], q[:, :_T]


def _pair_cm(sa, ca, qa, sb, cb, qb):
    sumy = sa + sb
    sxy = (ca - 15.5 * sa) + (cb + 0.5 * sb)
    syy = (qa + qb) - sumy * sumy * (1.0 / 32.0)
    r2 = (sxy * sxy) / (_SXX * syy)
    return (r2 >= _R2T).astype(jnp.int32)


def _lane_shift_right(x, sh, fill):
    pad = jnp.full((x.shape[0], sh), fill, x.dtype)
    return jnp.concatenate([pad, x[:, :-sh]], axis=1)


def _kernel(x_ref, w_ref, b_ref, out_ref, wbuf, svbuf, wtmp, svtmp,
            wsem, ssem):
    x = x_ref[0]
    xb = x.astype(jnp.bfloat16).astype(jnp.float32)
    xpad = jnp.pad(xb, ((0, 0), (120, 0)))

    stats = []
    tails = []
    for di, d in enumerate(_DIL):
        o = jnp.zeros((_C, _S), jnp.float32)
        for k in range(_P):
            off = 120 - (15 - k) * d
            wk = w_ref[di, k].astype(jnp.bfloat16).astype(jnp.float32)
            o = o + wk * xpad[:, off:off + _S]
        o = o + b_ref[di]
        oc = o - jnp.mean(o, axis=1, keepdims=True)
        stats.append(_win16(oc))
        tails.append((o[:, _T - 1:], o[:, :_P]))

        pieces = [jnp.pad(o[:, k:], ((0, 0), (0, k))) for k in range(_P)]
        wt = jnp.concatenate(pieces, axis=0)
        wbuf[pl.ds(di * _T, _T), :] = jnp.transpose(wt)[: _T]

    cm_parts = []
    for di in range(4):
        sa, ca, qa = stats[di]
        interior = _pair_cm(sa[:, : _T - 1], ca[:, : _T - 1], qa[:, : _T - 1],
                            sa[:, 1:], ca[:, 1:], qa[:, 1:])
        if di < 3:
            y = jnp.concatenate([tails[di][0], tails[di + 1][1]], axis=1)
            ym = y - jnp.mean(y, axis=1, keepdims=True)
            xm = (jax.lax.broadcasted_iota(jnp.int32, (_C, 2 * _P), 1)
                  .astype(jnp.float32) - 15.5)
            sxy = jnp.sum(ym * xm, axis=1, keepdims=True)
            syy = jnp.sum(ym * ym, axis=1, keepdims=True)
            r2 = (sxy * sxy) / (_SXX * syy)
            bnd = (r2 >= _R2T).astype(jnp.int32)
        else:
            bnd = jnp.zeros((_C, 1), jnp.int32)
        cm_parts.extend([interior, bnd])
    cm = jnp.concatenate(cm_parts, axis=1)
    cm = jnp.pad(cm, ((0, 0), (0, _NL - _N)))

    lane = jax.lax.broadcasted_iota(jnp.int32, (_C, _NL), 1)
    z = jnp.where(cm == 1, -1, lane)
    sh = 1
    for _ in range(13):
        z = jnp.maximum(z, _lane_shift_right(z, sh, -1))
        sh *= 2
    merge = cm * (((lane - z) & 1))
    valid = (1 - _lane_shift_right(merge, 1, 0)) * (lane < _N).astype(jnp.int32)
    s = 1 - valid
    sh = 1
    for _ in range(13):
        s = s + _lane_shift_right(s, sh, 0)
        sh *= 2
    km = jnp.max(jnp.where(valid == 1, s, 0))

    sv = (s << 2) | (merge << 1) | valid
    svbuf[pl.ds(0, _NL), :] = jnp.transpose(sv)

    a0 = (valid * jnp.where(s == 0, 1, 0)).astype(jnp.float32)
    b0 = a0 * merge.astype(jnp.float32)
    a0t = jnp.transpose(a0)
    b0t = jnp.transpose(b0)
    eye = (jnp.equal(jax.lax.broadcasted_iota(jnp.int32, (_C, 128), 1) & 7,
                     jax.lax.broadcasted_iota(jnp.int32, (_C, 128), 0))
           .astype(jnp.float32))
    wbuf[pl.ds(_N, _NL - _N), :] = jnp.zeros((_NL - _N, 128), jnp.float32)

    @pl.when(km > _NL - _N)
    def _zero_far_pad():
        svbuf[pl.ds(_NL, _PR - _NL), :] = jnp.zeros(
            (_PR - _NL, _C), jnp.int32)

    _CH = 1024
    _chunks = [(c * _CH, min(_CH, _N - c * _CH))
               for c in range((_N + _CH - 1) // _CH)]

    def emit(svk8, we, sz, k, r0, add):
        svk = jnp.concatenate([svk8] * 16, axis=1)
        w1 = we[0:sz]
        w2 = we[1:sz + 1]
        mask = jnp.logical_and((svk & 1) == 1, (svk >> 2) == k)
        val = jnp.where((svk & 2) == 2, 0.5 * (w1 + w2), w1)
        t = jnp.where(mask, val, 0.0)
        if add:
            out_ref[0, pl.ds(r0, sz), :] += t
        else:
            out_ref[0, pl.ds(r0, sz), :] = t

    for r0, sz in _chunks:
        we = wbuf[pl.ds(r0, sz + 8), :]
        w1 = we[0:sz]
        w2 = we[1:sz + 1]
        a = jnp.dot(a0t[r0:r0 + sz], eye,
                    preferred_element_type=jnp.float32)
        b = jnp.dot(b0t[r0:r0 + sz], eye,
                    preferred_element_type=jnp.float32)
        out_ref[0, pl.ds(r0, sz), :] = a * w1 + (0.5 * b) * (w2 - w1)

    def body(k, _):
        for r0, sz in _chunks:
            cw = pltpu.make_async_copy(
                wbuf.at[pl.ds(k + r0, sz + 8)],
                wtmp.at[pl.ds(0, sz + 8)], wsem)
            cs = pltpu.make_async_copy(
                svbuf.at[pl.ds(k + r0, sz)],
                svtmp.at[pl.ds(0, sz)], ssem)
            cw.start()
            cs.start()
            cw.wait()
            cs.wait()
            we = wtmp[pl.ds(0, sz + 8), :]
            svk8 = svtmp[pl.ds(0, sz), :]
            emit(svk8, we, sz, k, r0, True)
        return 0

    jax.lax.fori_loop(1, km + 1, body, 0)


@jax.jit
def kernel(seasonal, conv_w, conv_b):
    xt = seasonal.transpose(0, 2, 1)
    out = pl.pallas_call(
        _kernel,
        grid=(32,),
        in_specs=[
            pl.BlockSpec((1, _C, _S), lambda b: (b, 0, 0)),
            pl.BlockSpec(memory_space=pltpu.SMEM),
            pl.BlockSpec(memory_space=pltpu.SMEM),
        ],
        out_specs=pl.BlockSpec((1, _N, 128), lambda b: (b, 0, 0)),
        out_shape=jax.ShapeDtypeStruct((32, _N, 128), jnp.float32),
        scratch_shapes=[
            pltpu.VMEM((_PR, 128), jnp.float32),
            pltpu.VMEM((_PR, _C), jnp.int32),
            pltpu.VMEM((1032, 128), jnp.float32),
            pltpu.VMEM((1024, _C), jnp.int32),
            pltpu.SemaphoreType.DMA,
            pltpu.SemaphoreType.DMA,
        ],
        compiler_params=pltpu.CompilerParams(
            dimension_semantics=("parallel",),
        ),
    )(xt, conv_w, conv_b)
    return out.reshape(32, _N, _P, _C)
